# Initial kernel scaffold; baseline (speedup 1.0000x reference)
import jax
import jax.numpy as jnp
from jax.experimental import pallas as pl


def kernel(x_d, x_t, edge_index_dd, edge_index_tt, edge_index_dt, edge_index_td, W_dd, b_dd, W_tt, b_tt, W_dt, b_dt):
    raise NotImplementedError("write your pallas kernel here")


# speedup vs baseline: 4.0652x; 4.0652x over previous
import functools

import jax
import jax.numpy as jnp
from jax import lax
from jax.experimental import pallas as pl
from jax.experimental.pallas import tpu as pltpu
from jax.experimental.pallas import tpu_sc as plsc

N = 25000
DIN = 128
E = 200000
NTILE = 16
NCORE = 2
NPAD = 25088
EPAD = 204800
CHUNK = 128
NCHUNK = EPAD // NTILE // CHUNK
STRIPE = NPAD // NTILE


def _bincount8_kernel(idx_hbm, out_hbm, idxm, ones_v, zb, h0, h1, h2, h3):
    c = lax.axis_index("c")
    s = lax.axis_index("s")
    hists = [h0, h1, h2, h3]
    for i in range(CHUNK // 16):
        ones_v[pl.ds(16 * i, 16)] = jnp.ones((16,), jnp.float32)
    for i in range(STRIPE // 16):
        zb[pl.ds(16 * i, 16)] = jnp.zeros((16,), jnp.float32)
    for a in range(4):
        pltpu.sync_copy(zb, hists[a].at[pl.ds(s * STRIPE, STRIPE)])
    plsc.subcore_barrier()
    for a in range(4):
        arr = 4 * c + a
        pltpu.sync_copy(idx_hbm.at[arr * NTILE + s], idxm)

        def body(j, carry, _h=hists[a]):
            pltpu.sync_copy(ones_v, _h.at[idxm.at[j]], add=True)
            return carry

        lax.fori_loop(0, NCHUNK, body, None)
    plsc.subcore_barrier()
    for a in range(4):
        arr = 4 * c + a
        pltpu.sync_copy(hists[a].at[pl.ds(s * STRIPE, STRIPE)], zb)
        pltpu.sync_copy(zb, out_hbm.at[pl.ds(arr * NPAD + s * STRIPE, STRIPE)])


def _bincount8(idx8):
    idx_rows = idx8.reshape(8 * NTILE, NCHUNK, CHUNK)
    mesh = plsc.VectorSubcoreMesh(core_axis_name="c", subcore_axis_name="s")
    f = functools.partial(
        pl.kernel,
        mesh=mesh,
        out_type=jax.ShapeDtypeStruct((8 * NPAD,), jnp.float32),
        scratch_types=[
            pltpu.VMEM((NCHUNK, CHUNK), jnp.int32),
            pltpu.VMEM((CHUNK,), jnp.float32),
            pltpu.VMEM((STRIPE,), jnp.float32),
            pltpu.VMEM_SHARED((NPAD,), jnp.float32),
            pltpu.VMEM_SHARED((NPAD,), jnp.float32),
            pltpu.VMEM_SHARED((NPAD,), jnp.float32),
            pltpu.VMEM_SHARED((NPAD,), jnp.float32),
        ],
    )(_bincount8_kernel)
    return f(idx_rows).reshape(8, NPAD)


DSTAGE = 56
ACHUNK = 100
ANCH = EPAD // NTILE // ACHUNK
AHALF = ANCH // 2


def _agg_kernel(z_hbm, idxs_hbm, idxd_hbm, out_hbm,
                idxs, idxd, buf0, buf1, stage, acc, sem0, sem1):
    c = lax.axis_index("c")
    s = lax.axis_index("s")

    def zrow(r, carry):
        for col in range(4):
            stage[r, pl.ds(col * 16, 16)] = jnp.zeros((16,), jnp.float32)
        return carry

    for rel in range(4):
        lax.fori_loop(0, DSTAGE, zrow, None)
        for k in range(STRIPE // DSTAGE):
            pltpu.sync_copy(stage,
                            acc.at[pl.ds(s * STRIPE + k * DSTAGE, DSTAGE), :])
        plsc.subcore_barrier()

        def gather(j, buf, sem):
            return pltpu.make_async_copy(z_hbm.at[idxs.at[j]], buf, sem)

        for h in range(2):
            pltpu.sync_copy(
                idxs_hbm.at[((rel * 2 + c) * NTILE + s) * 2 + h], idxs)
            pltpu.sync_copy(idxd_hbm.at[(rel * NTILE + s) * 2 + h], idxd)
            gather(0, buf0, sem0).start()
            gather(1, buf1, sem1).start()

            def body(g, carry):
                j = 2 * g
                gather(j, buf0, sem0).wait()
                pltpu.sync_copy(buf0, acc.at[idxd.at[j]], add=True)

                @pl.when(g < AHALF // 2 - 1)
                def _():
                    gather(j + 2, buf0, sem0).start()

                gather(j + 1, buf1, sem1).wait()
                pltpu.sync_copy(buf1, acc.at[idxd.at[j + 1]], add=True)

                @pl.when(g < AHALF // 2 - 1)
                def _():
                    gather(j + 3, buf1, sem1).start()
                return carry

            lax.fori_loop(0, AHALF // 2, body, None)
        plsc.subcore_barrier()
        for k in range(STRIPE // DSTAGE):
            pltpu.sync_copy(acc.at[pl.ds(s * STRIPE + k * DSTAGE, DSTAGE), :],
                            stage)
            pltpu.sync_copy(
                stage,
                out_hbm.at[pl.ds((rel * 2 + c) * NPAD + s * STRIPE + k * DSTAGE,
                                 DSTAGE), :])


def _aggregate(z2, idx_src, idx_dst):
    mesh = plsc.VectorSubcoreMesh(core_axis_name="c", subcore_axis_name="s")
    f = functools.partial(
        pl.kernel,
        mesh=mesh,
        out_type=jax.ShapeDtypeStruct((8 * NPAD, 64), jnp.float32),
        scratch_types=[
            pltpu.VMEM((AHALF, ACHUNK), jnp.int32),
            pltpu.VMEM((AHALF, ACHUNK), jnp.int32),
            pltpu.VMEM((ACHUNK, 64), jnp.float32),
            pltpu.VMEM((ACHUNK, 64), jnp.float32),
            pltpu.VMEM((DSTAGE, 64), jnp.float32),
            pltpu.VMEM_SHARED((NPAD, 64), jnp.float32),
            pltpu.SemaphoreType.DMA,
            pltpu.SemaphoreType.DMA,
        ],
        compiler_params=pltpu.CompilerParams(use_tc_tiling_on_sc=False),
    )(_agg_kernel)
    return f(z2, idx_src, idx_dst)


def kernel(x_d, x_t, edge_index_dd, edge_index_tt, edge_index_dt,
           edge_index_td, W_dd, b_dd, W_tt, b_tt, W_dt, b_dt):
    srcs = [edge_index_dd[0], edge_index_td[0], edge_index_tt[0],
            edge_index_dt[0]]
    dsts = [edge_index_dd[1], edge_index_td[1], edge_index_tt[1],
            edge_index_dt[1]]
    npad_e = EPAD - E
    pad_idx = (N + (jnp.arange(npad_e) % 64)).astype(jnp.int32)
    srcs_p = [jnp.concatenate([s, pad_idx]) for s in srcs]
    dsts_p = [jnp.concatenate([d, pad_idx]) for d in dsts]

    idx8 = jnp.stack(srcs_p + dsts_p)
    hist = _bincount8(idx8)
    deg_src = hist[0:4, :N]
    deg_dst = hist[4:8, :N]

    xs = [x_d, x_t, x_t, x_d]
    Ws = [W_dd, W_dt, W_tt, W_dt]
    bs = [b_dd, b_dt, b_tt, b_dt]
    zr = []
    for r in range(4):
        sc = lax.rsqrt(jnp.clip(deg_src[r], 1.0))
        z = (xs[r] * sc[:, None]) @ Ws[r]
        zr.append(jnp.pad(z, ((0, NPAD - N), (0, 0))))
    z2 = jnp.stack(zr).reshape(8 * NPAD, 64)

    idx_src = jnp.stack(
        [jnp.stack([r * 2 * NPAD + 2 * srcs_p[r] + c for c in range(2)])
         for r in range(4)]).reshape(4 * 2 * NTILE * 2, AHALF, ACHUNK)
    idx_dst = jnp.stack(dsts_p).reshape(4 * NTILE * 2, AHALF, ACHUNK)
    agg = _aggregate(z2, idx_src, idx_dst).reshape(4, 2, NPAD, 64)

    outs = []
    for r in range(4):
        a = jnp.concatenate([agg[r, 0, :N], agg[r, 1, :N]], axis=1)
        outs.append(a * lax.rsqrt(jnp.clip(deg_dst[r], 1.0))[:, None] + bs[r])
    h_d = outs[0] + outs[1]
    h_t = outs[2] + outs[3]
    return (h_d, h_t)


# speedup vs baseline: 4.0886x; 1.0058x over previous
import functools

import jax
import jax.numpy as jnp
from jax import lax
from jax.experimental import pallas as pl
from jax.experimental.pallas import tpu as pltpu
from jax.experimental.pallas import tpu_sc as plsc

N = 25000
DIN = 128
E = 200000
NTILE = 16
NCORE = 2
NPAD = 25088
EPAD = 204800
CHUNK = 128
NCHUNK = EPAD // NTILE // CHUNK
STRIPE = NPAD // NTILE


def _bincount8_kernel(idx_hbm, out_hbm, idxm, ones_v, zb, h0, h1, h2, h3):
    c = lax.axis_index("c")
    s = lax.axis_index("s")
    hists = [h0, h1, h2, h3]
    for i in range(CHUNK // 16):
        ones_v[pl.ds(16 * i, 16)] = jnp.ones((16,), jnp.float32)
    for i in range(STRIPE // 16):
        zb[pl.ds(16 * i, 16)] = jnp.zeros((16,), jnp.float32)
    for a in range(4):
        pltpu.sync_copy(zb, hists[a].at[pl.ds(s * STRIPE, STRIPE)])
    plsc.subcore_barrier()
    for a in range(4):
        arr = 4 * c + a
        pltpu.sync_copy(idx_hbm.at[arr * NTILE + s], idxm)

        def body(j, carry, _h=hists[a]):
            pltpu.sync_copy(ones_v, _h.at[idxm.at[j]], add=True)
            return carry

        lax.fori_loop(0, NCHUNK, body, None)
    plsc.subcore_barrier()
    for a in range(4):
        arr = 4 * c + a
        pltpu.sync_copy(hists[a].at[pl.ds(s * STRIPE, STRIPE)], zb)
        pltpu.sync_copy(zb, out_hbm.at[pl.ds(arr * NPAD + s * STRIPE, STRIPE)])


def _bincount8(idx8):
    idx_rows = idx8.reshape(8 * NTILE, NCHUNK, CHUNK)
    mesh = plsc.VectorSubcoreMesh(core_axis_name="c", subcore_axis_name="s")
    f = functools.partial(
        pl.kernel,
        mesh=mesh,
        out_type=jax.ShapeDtypeStruct((8 * NPAD,), jnp.float32),
        scratch_types=[
            pltpu.VMEM((NCHUNK, CHUNK), jnp.int32),
            pltpu.VMEM((CHUNK,), jnp.float32),
            pltpu.VMEM((STRIPE,), jnp.float32),
            pltpu.VMEM_SHARED((NPAD,), jnp.float32),
            pltpu.VMEM_SHARED((NPAD,), jnp.float32),
            pltpu.VMEM_SHARED((NPAD,), jnp.float32),
            pltpu.VMEM_SHARED((NPAD,), jnp.float32),
        ],
    )(_bincount8_kernel)
    return f(idx_rows).reshape(8, NPAD)


DSTAGE = 56
ACHUNK = 100
ANCH = EPAD // NTILE // ACHUNK
AHALF = ANCH // 2


def _agg_kernel(z_hbm, idxs_hbm, idxd_hbm, out_hbm,
                idxs, idxd, buf0, buf1, stage, acc, sem0, sem1):
    c = lax.axis_index("c")
    s = lax.axis_index("s")

    def zrow(r, carry):
        for col in range(4):
            stage[r, pl.ds(col * 16, 16)] = jnp.zeros((16,), jnp.float32)
        return carry

    for rel in range(4):
        lax.fori_loop(0, DSTAGE, zrow, None)
        for k in range(STRIPE // DSTAGE):
            pltpu.sync_copy(stage,
                            acc.at[pl.ds(s * STRIPE + k * DSTAGE, DSTAGE), :])
        plsc.subcore_barrier()

        def gather(j, buf, sem):
            return pltpu.make_async_copy(z_hbm.at[idxs.at[j]], buf, sem)

        for h in range(2):
            pltpu.sync_copy(
                idxs_hbm.at[((rel * 2 + c) * NTILE + s) * 2 + h], idxs)
            pltpu.sync_copy(idxd_hbm.at[(rel * NTILE + s) * 2 + h], idxd)
            gather(0, buf0, sem0).start()
            gather(1, buf1, sem1).start()

            def body(g, carry):
                j = 2 * g
                gather(j, buf0, sem0).wait()
                pltpu.sync_copy(buf0, acc.at[idxd.at[j]], add=True)

                @pl.when(g < AHALF // 2 - 1)
                def _():
                    gather(j + 2, buf0, sem0).start()

                gather(j + 1, buf1, sem1).wait()
                pltpu.sync_copy(buf1, acc.at[idxd.at[j + 1]], add=True)

                @pl.when(g < AHALF // 2 - 1)
                def _():
                    gather(j + 3, buf1, sem1).start()
                return carry

            lax.fori_loop(0, AHALF // 2, body, None)
        plsc.subcore_barrier()
        for k in range(STRIPE // DSTAGE):
            pltpu.sync_copy(acc.at[pl.ds(s * STRIPE + k * DSTAGE, DSTAGE), :],
                            stage)
            pltpu.sync_copy(
                stage,
                out_hbm.at[pl.ds((rel * 2 + c) * NPAD + s * STRIPE + k * DSTAGE,
                                 DSTAGE), :])


def _aggregate(z2, idx_src, idx_dst):
    mesh = plsc.VectorSubcoreMesh(core_axis_name="c", subcore_axis_name="s")
    f = functools.partial(
        pl.kernel,
        mesh=mesh,
        out_type=jax.ShapeDtypeStruct((8 * NPAD, 64), jnp.float32),
        scratch_types=[
            pltpu.VMEM((AHALF, ACHUNK), jnp.int32),
            pltpu.VMEM((AHALF, ACHUNK), jnp.int32),
            pltpu.VMEM((ACHUNK, 64), jnp.float32),
            pltpu.VMEM((ACHUNK, 64), jnp.float32),
            pltpu.VMEM((DSTAGE, 64), jnp.float32),
            pltpu.VMEM_SHARED((NPAD, 64), jnp.float32),
            pltpu.SemaphoreType.DMA,
            pltpu.SemaphoreType.DMA,
        ],
        compiler_params=pltpu.CompilerParams(use_tc_tiling_on_sc=False),
    )(_agg_kernel)
    return f(z2, idx_src, idx_dst)


BR = 256
BRD = 200
_SRCTYPE = (0, 1, 1, 0)


def _z_kernel(x_ref, deg_ref, w_ref, out_ref):
    for rel in range(4):
        h = jnp.dot(x_ref[_SRCTYPE[rel]], w_ref[rel],
                    preferred_element_type=jnp.float32)
        sc = lax.rsqrt(jnp.maximum(deg_ref[:, rel:rel + 1], 1.0))
        out_ref[rel] = h * sc


def _z_transform(x_pad, deg_srcT, W_all):
    return pl.pallas_call(
        _z_kernel,
        grid=(NPAD // BR,),
        in_specs=[
            pl.BlockSpec((2, BR, DIN), lambda rb: (0, rb, 0)),
            pl.BlockSpec((BR, 4), lambda rb: (rb, 0)),
            pl.BlockSpec((4, DIN, DIN), lambda rb: (0, 0, 0)),
        ],
        out_specs=pl.BlockSpec((4, BR, DIN), lambda rb: (0, rb, 0)),
        out_shape=jax.ShapeDtypeStruct((4, NPAD, DIN), jnp.float32),
    )(x_pad, deg_srcT, W_all)


def _combine_kernel(agg_ref, deg_ref, bsum_ref, hd_ref, ht_ref):
    s = [lax.rsqrt(jnp.maximum(deg_ref[:, r:r + 1], 1.0)) for r in range(4)]
    hd_ref[...] = jnp.concatenate(
        [agg_ref[0, 0] * s[0] + agg_ref[1, 0] * s[1],
         agg_ref[0, 1] * s[0] + agg_ref[1, 1] * s[1]], axis=1) + bsum_ref[0:1]
    ht_ref[...] = jnp.concatenate(
        [agg_ref[2, 0] * s[2] + agg_ref[3, 0] * s[3],
         agg_ref[2, 1] * s[2] + agg_ref[3, 1] * s[3]], axis=1) + bsum_ref[1:2]


def _combine(agg, deg_dstT, bsum):
    return pl.pallas_call(
        _combine_kernel,
        grid=(N // BRD,),
        in_specs=[
            pl.BlockSpec((4, 2, BRD, 64), lambda rb: (0, 0, rb, 0)),
            pl.BlockSpec((BRD, 4), lambda rb: (rb, 0)),
            pl.BlockSpec((2, DIN), lambda rb: (0, 0)),
        ],
        out_specs=[pl.BlockSpec((BRD, DIN), lambda rb: (rb, 0)),
                   pl.BlockSpec((BRD, DIN), lambda rb: (rb, 0))],
        out_shape=[jax.ShapeDtypeStruct((N, DIN), jnp.float32),
                   jax.ShapeDtypeStruct((N, DIN), jnp.float32)],
    )(agg, deg_dstT, bsum)


def kernel(x_d, x_t, edge_index_dd, edge_index_tt, edge_index_dt,
           edge_index_td, W_dd, b_dd, W_tt, b_tt, W_dt, b_dt):
    srcs = [edge_index_dd[0], edge_index_td[0], edge_index_tt[0],
            edge_index_dt[0]]
    dsts = [edge_index_dd[1], edge_index_td[1], edge_index_tt[1],
            edge_index_dt[1]]
    npad_e = EPAD - E
    pad_idx = (N + (jnp.arange(npad_e) % 64)).astype(jnp.int32)
    srcs_p = [jnp.concatenate([s, pad_idx]) for s in srcs]
    dsts_p = [jnp.concatenate([d, pad_idx]) for d in dsts]

    idx8 = jnp.stack(srcs_p + dsts_p)
    hist = _bincount8(idx8)

    x_pad = jnp.stack([jnp.pad(x_d, ((0, NPAD - N), (0, 0))),
                       jnp.pad(x_t, ((0, NPAD - N), (0, 0)))])
    W_all = jnp.stack([W_dd, W_dt, W_tt, W_dt])
    deg_srcT = hist[0:4].T
    z2 = _z_transform(x_pad, deg_srcT, W_all).reshape(8 * NPAD, 64)

    idx_src = jnp.stack(
        [jnp.stack([r * 2 * NPAD + 2 * srcs_p[r] + c for c in range(2)])
         for r in range(4)]).reshape(4 * 2 * NTILE * 2, AHALF, ACHUNK)
    idx_dst = jnp.stack(dsts_p).reshape(4 * NTILE * 2, AHALF, ACHUNK)
    agg = _aggregate(z2, idx_src, idx_dst).reshape(4, 2, NPAD, 64)

    deg_dstT = hist[4:8].T
    bsum = jnp.stack([b_dd + b_dt, b_tt + b_dt])
    h_d, h_t = _combine(agg, deg_dstT, bsum)
    return (h_d, h_t)


# speedup vs baseline: 4.8282x; 1.1809x over previous
import functools

import jax
import jax.numpy as jnp
from jax import lax
from jax.experimental import pallas as pl
from jax.experimental.pallas import tpu as pltpu
from jax.experimental.pallas import tpu_sc as plsc

N = 25000
DIN = 128
E = 200000
NTILE = 16
NCORE = 2
NPAD = 25088
EPAD = 204800
CHUNK = 128
NCHUNK = EPAD // NTILE // CHUNK
STRIPE = NPAD // NTILE


def _bincount8_kernel(e0, e1, e2, e3, out_hbm, idxm, ones_v, zb,
                      h0, h1, h2, h3):
    c = lax.axis_index("c")
    s = lax.axis_index("s")
    hists = [h0, h1, h2, h3]
    for i in range(CHUNK // 16):
        ones_v[pl.ds(16 * i, 16)] = jnp.ones((16,), jnp.float32)
    for i in range(STRIPE // 16):
        zb[pl.ds(16 * i, 16)] = jnp.zeros((16,), jnp.float32)
    for a in range(4):
        pltpu.sync_copy(zb, hists[a].at[pl.ds(s * STRIPE, STRIPE)])
    plsc.subcore_barrier()
    for a, e in enumerate((e0, e1, e2, e3)):
        pltpu.sync_copy(e.at[c * NTILE + s], idxm)

        def body(j, carry, _h=hists[a]):
            pltpu.sync_copy(ones_v, _h.at[idxm.at[j]], add=True)
            return carry

        lax.fori_loop(0, NCHUNK, body, None)
    plsc.subcore_barrier()
    for a in range(4):
        arr = 4 * c + a
        pltpu.sync_copy(hists[a].at[pl.ds(s * STRIPE, STRIPE)], zb)
        pltpu.sync_copy(zb, out_hbm.at[pl.ds(arr * NPAD + s * STRIPE, STRIPE)])


def _bincount8(e_pads):
    mesh = plsc.VectorSubcoreMesh(core_axis_name="c", subcore_axis_name="s")
    f = functools.partial(
        pl.kernel,
        mesh=mesh,
        out_type=jax.ShapeDtypeStruct((8 * NPAD,), jnp.float32),
        scratch_types=[
            pltpu.VMEM((NCHUNK, CHUNK), jnp.int32),
            pltpu.VMEM((CHUNK,), jnp.float32),
            pltpu.VMEM((STRIPE,), jnp.float32),
            pltpu.VMEM_SHARED((NPAD,), jnp.float32),
            pltpu.VMEM_SHARED((NPAD,), jnp.float32),
            pltpu.VMEM_SHARED((NPAD,), jnp.float32),
            pltpu.VMEM_SHARED((NPAD,), jnp.float32),
        ],
    )(_bincount8_kernel)
    return f(*e_pads).reshape(8, NPAD)


DSTAGE = 56
ACHUNK = 100
ANCH = EPAD // NTILE // ACHUNK
AHALF = ANCH // 2


def _agg_kernel(z_hbm, idxs_hbm, idxd_hbm, out_hbm,
                idxs, idxd, buf0, buf1, stage, acc, sem0, sem1):
    c = lax.axis_index("c")
    s = lax.axis_index("s")

    def zrow(r, carry):
        for col in range(4):
            stage[r, pl.ds(col * 16, 16)] = jnp.zeros((16,), jnp.float32)
        return carry

    for rel in range(4):
        lax.fori_loop(0, DSTAGE, zrow, None)
        for k in range(STRIPE // DSTAGE):
            pltpu.sync_copy(stage,
                            acc.at[pl.ds(s * STRIPE + k * DSTAGE, DSTAGE), :])
        plsc.subcore_barrier()

        def gather(j, buf, sem):
            return pltpu.make_async_copy(z_hbm.at[idxs.at[j]], buf, sem)

        for h in range(2):
            pltpu.sync_copy(
                idxs_hbm.at[((rel * 2 + c) * NTILE + s) * 2 + h], idxs)
            pltpu.sync_copy(idxd_hbm.at[(rel * NTILE + s) * 2 + h], idxd)
            gather(0, buf0, sem0).start()
            gather(1, buf1, sem1).start()

            def body(g, carry):
                j = 2 * g
                gather(j, buf0, sem0).wait()
                pltpu.sync_copy(buf0, acc.at[idxd.at[j]], add=True)

                @pl.when(g < AHALF // 2 - 1)
                def _():
                    gather(j + 2, buf0, sem0).start()

                gather(j + 1, buf1, sem1).wait()
                pltpu.sync_copy(buf1, acc.at[idxd.at[j + 1]], add=True)

                @pl.when(g < AHALF // 2 - 1)
                def _():
                    gather(j + 3, buf1, sem1).start()
                return carry

            lax.fori_loop(0, AHALF // 2, body, None)
        plsc.subcore_barrier()
        for k in range(STRIPE // DSTAGE):
            pltpu.sync_copy(acc.at[pl.ds(s * STRIPE + k * DSTAGE, DSTAGE), :],
                            stage)
            pltpu.sync_copy(
                stage,
                out_hbm.at[rel, pl.ds(s * STRIPE + k * DSTAGE, DSTAGE),
                           pl.ds(c * 64, 64)])


def _aggregate(z2, idx_src, idx_dst):
    mesh = plsc.VectorSubcoreMesh(core_axis_name="c", subcore_axis_name="s")
    f = functools.partial(
        pl.kernel,
        mesh=mesh,
        out_type=jax.ShapeDtypeStruct((4, NPAD, 128), jnp.float32),
        scratch_types=[
            pltpu.VMEM((AHALF, ACHUNK), jnp.int32),
            pltpu.VMEM((AHALF, ACHUNK), jnp.int32),
            pltpu.VMEM((ACHUNK, 64), jnp.float32),
            pltpu.VMEM((ACHUNK, 64), jnp.float32),
            pltpu.VMEM((DSTAGE, 64), jnp.float32),
            pltpu.VMEM_SHARED((NPAD, 64), jnp.float32),
            pltpu.SemaphoreType.DMA,
            pltpu.SemaphoreType.DMA,
        ],
        compiler_params=pltpu.CompilerParams(use_tc_tiling_on_sc=False),
    )(_agg_kernel)
    return f(z2, idx_src, idx_dst)


BR = 200
BRD = 200
_SRCTYPE = (0, 1, 1, 0)


def _z_kernel(xd_ref, xt_ref, deg_ref, w_ref, out_ref):
    xs = (xd_ref, xt_ref)
    for rel in range(4):
        h = jnp.dot(xs[_SRCTYPE[rel]][...], w_ref[rel],
                    preferred_element_type=jnp.float32)
        sc = lax.rsqrt(jnp.maximum(deg_ref[:, rel:rel + 1], 1.0))
        out_ref[rel] = h * sc


def _z_transform(x_d, x_t, deg_srcT, W_all):
    return pl.pallas_call(
        _z_kernel,
        grid=(N // BR,),
        in_specs=[
            pl.BlockSpec((BR, DIN), lambda rb: (rb, 0)),
            pl.BlockSpec((BR, DIN), lambda rb: (rb, 0)),
            pl.BlockSpec((BR, 4), lambda rb: (rb, 0)),
            pl.BlockSpec((4, DIN, DIN), lambda rb: (0, 0, 0)),
        ],
        out_specs=pl.BlockSpec((4, BR, DIN), lambda rb: (0, rb, 0)),
        out_shape=jax.ShapeDtypeStruct((4, NPAD, DIN), jnp.float32),
    )(x_d, x_t, deg_srcT, W_all)


def _combine_kernel(agg_ref, deg_ref, bsum_ref, hd_ref, ht_ref):
    s = [lax.rsqrt(jnp.maximum(deg_ref[:, r:r + 1], 1.0)) for r in range(4)]
    hd_ref[...] = agg_ref[0] * s[0] + agg_ref[1] * s[1] + bsum_ref[0:1]
    ht_ref[...] = agg_ref[2] * s[2] + agg_ref[3] * s[3] + bsum_ref[1:2]


def _combine(agg, deg_dstT, bsum):
    return pl.pallas_call(
        _combine_kernel,
        grid=(N // BRD,),
        in_specs=[
            pl.BlockSpec((4, BRD, DIN), lambda rb: (0, rb, 0)),
            pl.BlockSpec((BRD, 4), lambda rb: (rb, 0)),
            pl.BlockSpec((2, DIN), lambda rb: (0, 0)),
        ],
        out_specs=[pl.BlockSpec((BRD, DIN), lambda rb: (rb, 0)),
                   pl.BlockSpec((BRD, DIN), lambda rb: (rb, 0))],
        out_shape=[jax.ShapeDtypeStruct((N, DIN), jnp.float32),
                   jax.ShapeDtypeStruct((N, DIN), jnp.float32)],
    )(agg, deg_dstT, bsum)


def kernel(x_d, x_t, edge_index_dd, edge_index_tt, edge_index_dt,
           edge_index_td, W_dd, b_dd, W_tt, b_tt, W_dt, b_dt):
    edges = [edge_index_dd, edge_index_td, edge_index_tt, edge_index_dt]
    npad_e = EPAD - E
    pad_idx = (N + (jnp.arange(npad_e) % 64)).astype(jnp.int32)
    pad2 = jnp.stack([pad_idx, pad_idx])
    e_pads = [jnp.concatenate([e, pad2], axis=1) for e in edges]

    hist = _bincount8(
        [e.reshape(2 * NTILE, NCHUNK, CHUNK) for e in e_pads])

    W_all = jnp.stack([W_dd, W_dt, W_tt, W_dt])
    deg_srcT = hist[0:4].T
    z = _z_transform(x_d, x_t, deg_srcT, W_all)
    z2 = z.reshape(8 * NPAD, 64)

    idx_src = jnp.stack(
        [jnp.stack([r * 2 * NPAD + 2 * e_pads[r][0] + c for c in range(2)])
         for r in range(4)]).reshape(4 * 2 * NTILE * 2, AHALF, ACHUNK)
    idx_dst = jnp.stack([e[1] for e in e_pads]).reshape(
        4 * NTILE * 2, AHALF, ACHUNK)
    agg = _aggregate(z2, idx_src, idx_dst)

    deg_dstT = hist[4:8].T
    bsum = jnp.stack([b_dd + b_dt, b_tt + b_dt])
    h_d, h_t = _combine(agg, deg_dstT, bsum)
    return (h_d, h_t)


# speedup vs baseline: 5.2208x; 1.0813x over previous
import functools

import jax
import jax.numpy as jnp
from jax import lax
from jax.experimental import pallas as pl
from jax.experimental.pallas import tpu as pltpu
from jax.experimental.pallas import tpu_sc as plsc

N = 25000
DIN = 128
E = 200000
NTILE = 16
NCORE = 2
NPAD = 25088
EPAD = 204800
CHUNK = 128
NCHUNK = EPAD // NTILE // CHUNK
STRIPE = NPAD // NTILE


def _bincount8_kernel(e0, e1, e2, e3, out_hbm, idxm, ones_v, zb,
                      h0, h1, h2, h3):
    c = lax.axis_index("c")
    s = lax.axis_index("s")
    hists = [h0, h1, h2, h3]
    for i in range(CHUNK // 16):
        ones_v[pl.ds(16 * i, 16)] = jnp.ones((16,), jnp.float32)
    for i in range(STRIPE // 16):
        zb[pl.ds(16 * i, 16)] = jnp.zeros((16,), jnp.float32)
    for a in range(4):
        pltpu.sync_copy(zb, hists[a].at[pl.ds(s * STRIPE, STRIPE)])
    plsc.subcore_barrier()
    for a, e in enumerate((e0, e1, e2, e3)):
        pltpu.sync_copy(e.at[c * NTILE + s], idxm)

        def body(j, carry, _h=hists[a]):
            pltpu.sync_copy(ones_v, _h.at[idxm.at[j]], add=True)
            return carry

        lax.fori_loop(0, NCHUNK, body, None)
    plsc.subcore_barrier()
    for a in range(4):
        arr = 4 * c + a
        pltpu.sync_copy(hists[a].at[pl.ds(s * STRIPE, STRIPE)], zb)
        pltpu.sync_copy(zb, out_hbm.at[pl.ds(arr * NPAD + s * STRIPE, STRIPE)])


def _bincount8(e_pads):
    mesh = plsc.VectorSubcoreMesh(core_axis_name="c", subcore_axis_name="s")
    f = functools.partial(
        pl.kernel,
        mesh=mesh,
        out_type=jax.ShapeDtypeStruct((8 * NPAD,), jnp.float32),
        scratch_types=[
            pltpu.VMEM((NCHUNK, CHUNK), jnp.int32),
            pltpu.VMEM((CHUNK,), jnp.float32),
            pltpu.VMEM((STRIPE,), jnp.float32),
            pltpu.VMEM_SHARED((NPAD,), jnp.float32),
            pltpu.VMEM_SHARED((NPAD,), jnp.float32),
            pltpu.VMEM_SHARED((NPAD,), jnp.float32),
            pltpu.VMEM_SHARED((NPAD,), jnp.float32),
        ],
    )(_bincount8_kernel)
    return f(*e_pads).reshape(8, NPAD)


DSTAGE = 56
ACHUNK = 100
ANCH = EPAD // NTILE // ACHUNK
AHALF = ANCH // 2


def _agg_kernel(z_hbm, idxs_hbm, idxd_hbm, out_hbm,
                idxs, idxd, buf0, buf1, stage, acc, sem0, sem1):
    c = lax.axis_index("c")
    s = lax.axis_index("s")

    def zrow(r, carry):
        for col in range(4):
            stage[r, pl.ds(col * 16, 16)] = jnp.zeros((16,), jnp.float32)
        return carry

    for rel in range(4):
        lax.fori_loop(0, DSTAGE, zrow, None)
        for k in range(STRIPE // DSTAGE):
            pltpu.sync_copy(stage,
                            acc.at[pl.ds(s * STRIPE + k * DSTAGE, DSTAGE), :])
        plsc.subcore_barrier()

        def gather(j, buf, sem):
            return pltpu.make_async_copy(z_hbm.at[idxs.at[j]], buf, sem)

        for h in range(2):
            pltpu.sync_copy(
                idxs_hbm.at[((rel * 2 + c) * NTILE + s) * 2 + h], idxs)
            pltpu.sync_copy(idxd_hbm.at[(rel * NTILE + s) * 2 + h], idxd)
            gather(0, buf0, sem0).start()
            gather(1, buf1, sem1).start()

            def body(g, carry):
                j = 2 * g
                gather(j, buf0, sem0).wait()
                pltpu.sync_copy(buf0, acc.at[idxd.at[j]], add=True)

                @pl.when(g < AHALF // 2 - 1)
                def _():
                    gather(j + 2, buf0, sem0).start()

                gather(j + 1, buf1, sem1).wait()
                pltpu.sync_copy(buf1, acc.at[idxd.at[j + 1]], add=True)

                @pl.when(g < AHALF // 2 - 1)
                def _():
                    gather(j + 3, buf1, sem1).start()
                return carry

            lax.fori_loop(0, AHALF // 2, body, None)
        plsc.subcore_barrier()
        for k in range(STRIPE // DSTAGE):
            pltpu.sync_copy(acc.at[pl.ds(s * STRIPE + k * DSTAGE, DSTAGE), :],
                            stage)
            pltpu.sync_copy(
                stage,
                out_hbm.at[rel, pl.ds(s * STRIPE + k * DSTAGE, DSTAGE),
                           pl.ds(c * 64, 64)])


def _aggregate(z2, idx_src, idx_dst):
    mesh = plsc.VectorSubcoreMesh(core_axis_name="c", subcore_axis_name="s")
    f = functools.partial(
        pl.kernel,
        mesh=mesh,
        out_type=jax.ShapeDtypeStruct((4, NPAD, 128), jnp.float32),
        scratch_types=[
            pltpu.VMEM((AHALF, ACHUNK), jnp.int32),
            pltpu.VMEM((AHALF, ACHUNK), jnp.int32),
            pltpu.VMEM((ACHUNK, 64), jnp.float32),
            pltpu.VMEM((ACHUNK, 64), jnp.float32),
            pltpu.VMEM((DSTAGE, 64), jnp.float32),
            pltpu.VMEM_SHARED((NPAD, 64), jnp.float32),
            pltpu.SemaphoreType.DMA,
            pltpu.SemaphoreType.DMA,
        ],
        compiler_params=pltpu.CompilerParams(use_tc_tiling_on_sc=False),
    )(_agg_kernel)
    return f(z2, idx_src, idx_dst)


BR = 1000
BRD = 200
_SRCTYPE = (0, 1, 1, 0)


def _z_kernel(xd_ref, xt_ref, deg_ref, w_ref, out_ref):
    xs = (xd_ref, xt_ref)
    for rel in range(4):
        h = jnp.dot(xs[_SRCTYPE[rel]][...], w_ref[rel],
                    preferred_element_type=jnp.float32)
        sc = lax.rsqrt(jnp.maximum(deg_ref[:, rel:rel + 1], 1.0))
        out_ref[rel] = h * sc


def _z_transform(x_d, x_t, deg_srcT, W_all):
    return pl.pallas_call(
        _z_kernel,
        grid=(N // BR,),
        in_specs=[
            pl.BlockSpec((BR, DIN), lambda rb: (rb, 0)),
            pl.BlockSpec((BR, DIN), lambda rb: (rb, 0)),
            pl.BlockSpec((BR, 4), lambda rb: (rb, 0)),
            pl.BlockSpec((4, DIN, DIN), lambda rb: (0, 0, 0)),
        ],
        out_specs=pl.BlockSpec((4, BR, DIN), lambda rb: (0, rb, 0)),
        out_shape=jax.ShapeDtypeStruct((4, NPAD, DIN), jnp.float32),
    )(x_d, x_t, deg_srcT, W_all)


def _combine_kernel(agg_ref, deg_ref, bsum_ref, hd_ref, ht_ref):
    s = [lax.rsqrt(jnp.maximum(deg_ref[:, r:r + 1], 1.0)) for r in range(4)]
    hd_ref[...] = agg_ref[0] * s[0] + agg_ref[1] * s[1] + bsum_ref[0:1]
    ht_ref[...] = agg_ref[2] * s[2] + agg_ref[3] * s[3] + bsum_ref[1:2]


def _combine(agg, deg_dstT, bsum):
    return pl.pallas_call(
        _combine_kernel,
        grid=(N // BRD,),
        in_specs=[
            pl.BlockSpec((4, BRD, DIN), lambda rb: (0, rb, 0)),
            pl.BlockSpec((BRD, 4), lambda rb: (rb, 0)),
            pl.BlockSpec((2, DIN), lambda rb: (0, 0)),
        ],
        out_specs=[pl.BlockSpec((BRD, DIN), lambda rb: (rb, 0)),
                   pl.BlockSpec((BRD, DIN), lambda rb: (rb, 0))],
        out_shape=[jax.ShapeDtypeStruct((N, DIN), jnp.float32),
                   jax.ShapeDtypeStruct((N, DIN), jnp.float32)],
    )(agg, deg_dstT, bsum)


def kernel(x_d, x_t, edge_index_dd, edge_index_tt, edge_index_dt,
           edge_index_td, W_dd, b_dd, W_tt, b_tt, W_dt, b_dt):
    edges = [edge_index_dd, edge_index_td, edge_index_tt, edge_index_dt]
    npad_e = EPAD - E
    pad_idx = (N + (jnp.arange(npad_e) % 64)).astype(jnp.int32)
    pad2 = jnp.stack([pad_idx, pad_idx])
    e_pads = [jnp.concatenate([e, pad2], axis=1) for e in edges]

    hist = _bincount8(
        [e.reshape(2 * NTILE, NCHUNK, CHUNK) for e in e_pads])

    W_all = jnp.stack([W_dd, W_dt, W_tt, W_dt])
    deg_srcT = hist[0:4].T
    z = _z_transform(x_d, x_t, deg_srcT, W_all)
    z2 = z.reshape(8 * NPAD, 64)

    idx_src = jnp.stack(
        [jnp.stack([r * 2 * NPAD + 2 * e_pads[r][0] + c for c in range(2)])
         for r in range(4)]).reshape(4 * 2 * NTILE * 2, AHALF, ACHUNK)
    idx_dst = jnp.stack([e[1] for e in e_pads]).reshape(
        4 * NTILE * 2, AHALF, ACHUNK)
    agg = _aggregate(z2, idx_src, idx_dst)

    deg_dstT = hist[4:8].T
    bsum = jnp.stack([b_dd + b_dt, b_tt + b_dt])
    h_d, h_t = _combine(agg, deg_dstT, bsum)
    return (h_d, h_t)


# speedup vs baseline: 5.4484x; 1.0436x over previous
import functools

import jax
import jax.numpy as jnp
from jax import lax
from jax.experimental import pallas as pl
from jax.experimental.pallas import tpu as pltpu
from jax.experimental.pallas import tpu_sc as plsc

N = 25000
DIN = 128
E = 200000
NTILE = 16
NCORE = 2
NPAD = 25088
EPAD = 204800
CHUNK = 128
NCHUNK = EPAD // NTILE // CHUNK
STRIPE = NPAD // NTILE


def _bincount8_kernel(e0, e1, e2, e3, out_hbm, idxm, ones_v, zb,
                      h0, h1, h2, h3):
    c = lax.axis_index("c")
    s = lax.axis_index("s")
    hists = [h0, h1, h2, h3]
    for i in range(CHUNK // 16):
        ones_v[pl.ds(16 * i, 16)] = jnp.ones((16,), jnp.float32)
    for i in range(STRIPE // 16):
        zb[pl.ds(16 * i, 16)] = jnp.zeros((16,), jnp.float32)
    for a in range(4):
        pltpu.sync_copy(zb, hists[a].at[pl.ds(s * STRIPE, STRIPE)])
    plsc.subcore_barrier()
    for a, e in enumerate((e0, e1, e2, e3)):
        pltpu.sync_copy(e.at[c * NTILE + s], idxm)

        def body(j, carry, _h=hists[a]):
            pltpu.sync_copy(ones_v, _h.at[idxm.at[j]], add=True)
            return carry

        lax.fori_loop(0, NCHUNK, body, None)
    plsc.subcore_barrier()
    for a in range(4):
        arr = 4 * c + a
        pltpu.sync_copy(hists[a].at[pl.ds(s * STRIPE, STRIPE)], zb)
        pltpu.sync_copy(zb, out_hbm.at[pl.ds(arr * NPAD + s * STRIPE, STRIPE)])


def _bincount8(e_pads):
    mesh = plsc.VectorSubcoreMesh(core_axis_name="c", subcore_axis_name="s")
    f = functools.partial(
        pl.kernel,
        mesh=mesh,
        out_type=jax.ShapeDtypeStruct((8 * NPAD,), jnp.float32),
        scratch_types=[
            pltpu.VMEM((NCHUNK, CHUNK), jnp.int32),
            pltpu.VMEM((CHUNK,), jnp.float32),
            pltpu.VMEM((STRIPE,), jnp.float32),
            pltpu.VMEM_SHARED((NPAD,), jnp.float32),
            pltpu.VMEM_SHARED((NPAD,), jnp.float32),
            pltpu.VMEM_SHARED((NPAD,), jnp.float32),
            pltpu.VMEM_SHARED((NPAD,), jnp.float32),
        ],
    )(_bincount8_kernel)
    return f(*e_pads).reshape(8, NPAD)


DSTAGE = 56
ACHUNK = 100
ANCH = EPAD // NTILE // ACHUNK
AHALF = ANCH // 2


def _agg_kernel(z_hbm, idxs_hbm, idxd_hbm, out_hbm,
                idxs, idxd, buf0, buf1, stage, acc, sem0, sem1):
    c = lax.axis_index("c")
    s = lax.axis_index("s")

    def zrow(r, carry):
        for col in range(4):
            stage[r, pl.ds(col * 16, 16)] = jnp.zeros((16,), jnp.float32)
        return carry

    lax.fori_loop(0, DSTAGE, zrow, None)
    for k in range(STRIPE // DSTAGE):
        pltpu.sync_copy(stage,
                        acc.at[pl.ds(s * STRIPE + k * DSTAGE, DSTAGE), :])
    plsc.subcore_barrier()

    def gather(j, buf, sem):
        return pltpu.make_async_copy(z_hbm.at[idxs.at[j]], buf, sem)

    for h in range(2):
        pltpu.sync_copy(idxs_hbm.at[(c * NTILE + s) * 2 + h], idxs)
        pltpu.sync_copy(idxd_hbm.at[s * 2 + h], idxd)
        gather(0, buf0, sem0).start()
        gather(1, buf1, sem1).start()

        def body(g, carry):
            j = 2 * g
            gather(j, buf0, sem0).wait()
            pltpu.sync_copy(buf0, acc.at[idxd.at[j]], add=True)

            @pl.when(g < AHALF // 2 - 1)
            def _():
                gather(j + 2, buf0, sem0).start()

            gather(j + 1, buf1, sem1).wait()
            pltpu.sync_copy(buf1, acc.at[idxd.at[j + 1]], add=True)

            @pl.when(g < AHALF // 2 - 1)
            def _():
                gather(j + 3, buf1, sem1).start()
            return carry

        lax.fori_loop(0, AHALF // 2, body, None)
    plsc.subcore_barrier()
    for k in range(STRIPE // DSTAGE):
        pltpu.sync_copy(acc.at[pl.ds(s * STRIPE + k * DSTAGE, DSTAGE), :],
                        stage)
        pltpu.sync_copy(
            stage,
            out_hbm.at[pl.ds(s * STRIPE + k * DSTAGE, DSTAGE),
                       pl.ds(c * 64, 64)])


def _aggregate_rel(z2, idx_src, idx_dst):
    mesh = plsc.VectorSubcoreMesh(core_axis_name="c", subcore_axis_name="s")
    f = functools.partial(
        pl.kernel,
        mesh=mesh,
        out_type=jax.ShapeDtypeStruct((NPAD, 128), jnp.float32),
        scratch_types=[
            pltpu.VMEM((AHALF, ACHUNK), jnp.int32),
            pltpu.VMEM((AHALF, ACHUNK), jnp.int32),
            pltpu.VMEM((ACHUNK, 64), jnp.float32),
            pltpu.VMEM((ACHUNK, 64), jnp.float32),
            pltpu.VMEM((DSTAGE, 64), jnp.float32),
            pltpu.VMEM_SHARED((NPAD, 64), jnp.float32),
            pltpu.SemaphoreType.DMA,
            pltpu.SemaphoreType.DMA,
        ],
        compiler_params=pltpu.CompilerParams(use_tc_tiling_on_sc=False),
    )(_agg_kernel)
    return f(z2, idx_src, idx_dst)


BR = 1000
BRD = 200
_SRCTYPE = (0, 1, 1, 0)


def _make_z_kernel(rel):
    def _z_kernel(x_ref, deg_ref, w_ref, out_ref):
        h = jnp.dot(x_ref[...], w_ref[...],
                    preferred_element_type=jnp.float32)
        sc = lax.rsqrt(jnp.maximum(deg_ref[:, rel:rel + 1], 1.0))
        out_ref[...] = h * sc
    return _z_kernel


def _z_transform_rel(rel, x, deg_srcT, W):
    return pl.pallas_call(
        _make_z_kernel(rel),
        grid=(N // BR,),
        in_specs=[
            pl.BlockSpec((BR, DIN), lambda rb: (rb, 0)),
            pl.BlockSpec((BR, 4), lambda rb: (rb, 0)),
            pl.BlockSpec((DIN, DIN), lambda rb: (0, 0)),
        ],
        out_specs=pl.BlockSpec((BR, DIN), lambda rb: (rb, 0)),
        out_shape=jax.ShapeDtypeStruct((NPAD, DIN), jnp.float32),
    )(x, deg_srcT, W)


def _combine_kernel(a0_ref, a1_ref, a2_ref, a3_ref, deg_ref, bsum_ref,
                    hd_ref, ht_ref):
    s = [lax.rsqrt(jnp.maximum(deg_ref[:, r:r + 1], 1.0)) for r in range(4)]
    hd_ref[...] = a0_ref[...] * s[0] + a1_ref[...] * s[1] + bsum_ref[0:1]
    ht_ref[...] = a2_ref[...] * s[2] + a3_ref[...] * s[3] + bsum_ref[1:2]


def _combine(aggs, deg_dstT, bsum):
    return pl.pallas_call(
        _combine_kernel,
        grid=(N // BRD,),
        in_specs=[
            pl.BlockSpec((BRD, DIN), lambda rb: (rb, 0)),
            pl.BlockSpec((BRD, DIN), lambda rb: (rb, 0)),
            pl.BlockSpec((BRD, DIN), lambda rb: (rb, 0)),
            pl.BlockSpec((BRD, DIN), lambda rb: (rb, 0)),
            pl.BlockSpec((BRD, 4), lambda rb: (rb, 0)),
            pl.BlockSpec((2, DIN), lambda rb: (0, 0)),
        ],
        out_specs=[pl.BlockSpec((BRD, DIN), lambda rb: (rb, 0)),
                   pl.BlockSpec((BRD, DIN), lambda rb: (rb, 0))],
        out_shape=[jax.ShapeDtypeStruct((N, DIN), jnp.float32),
                   jax.ShapeDtypeStruct((N, DIN), jnp.float32)],
    )(*aggs, deg_dstT, bsum)


def kernel(x_d, x_t, edge_index_dd, edge_index_tt, edge_index_dt,
           edge_index_td, W_dd, b_dd, W_tt, b_tt, W_dt, b_dt):
    edges = [edge_index_dd, edge_index_td, edge_index_tt, edge_index_dt]
    npad_e = EPAD - E
    pad_idx = (N + (jnp.arange(npad_e) % 64)).astype(jnp.int32)
    pad2 = jnp.stack([pad_idx, pad_idx])
    e_pads = [jnp.concatenate([e, pad2], axis=1) for e in edges]

    hist = _bincount8(
        [e.reshape(2 * NTILE, NCHUNK, CHUNK) for e in e_pads])

    xs = (x_d, x_t)
    Ws = (W_dd, W_dt, W_tt, W_dt)
    deg_srcT = hist[0:4].T
    aggs = []
    for r in range(4):
        z = _z_transform_rel(r, xs[_SRCTYPE[r]], deg_srcT, Ws[r])
        z2 = z.reshape(2 * NPAD, 64)
        idx_src = jnp.stack(
            [2 * e_pads[r][0] + c for c in range(2)]).reshape(
                2 * NTILE * 2, AHALF, ACHUNK)
        idx_dst = e_pads[r][1].reshape(NTILE * 2, AHALF, ACHUNK)
        aggs.append(_aggregate_rel(z2, idx_src, idx_dst))

    deg_dstT = hist[4:8].T
    bsum = jnp.stack([b_dd + b_dt, b_tt + b_dt])
    h_d, h_t = _combine(aggs, deg_dstT, bsum)
    return (h_d, h_t)


# speedup vs baseline: 6.3058x; 1.1574x over previous
import functools

import jax
import jax.numpy as jnp
from jax import lax
from jax.experimental import pallas as pl
from jax.experimental.pallas import tpu as pltpu
from jax.experimental.pallas import tpu_sc as plsc

N = 25000
DIN = 128
E = 200000
NTILE = 16
NCORE = 2
NPAD = 25088
EPAD = 204800
CHUNK = 128
NCHUNK = EPAD // NTILE // CHUNK
STRIPE = NPAD // NTILE


def _bincount8_kernel(e0, e1, e2, e3, out_hbm, idxm, ones_v, zb,
                      h0, h1, h2, h3):
    c = lax.axis_index("c")
    s = lax.axis_index("s")
    hists = [h0, h1, h2, h3]
    for i in range(CHUNK // 16):
        ones_v[pl.ds(16 * i, 16)] = jnp.ones((16,), jnp.float32)
    for i in range(STRIPE // 16):
        zb[pl.ds(16 * i, 16)] = jnp.zeros((16,), jnp.float32)
    for a in range(4):
        pltpu.sync_copy(zb, hists[a].at[pl.ds(s * STRIPE, STRIPE)])
    plsc.subcore_barrier()
    for a, e in enumerate((e0, e1, e2, e3)):
        pltpu.sync_copy(e.at[c * NTILE + s], idxm)

        def body(j, carry, _h=hists[a]):
            pltpu.sync_copy(ones_v, _h.at[idxm.at[j]], add=True)
            return carry

        lax.fori_loop(0, NCHUNK, body, None)
    plsc.subcore_barrier()
    for a in range(4):
        arr = 4 * c + a
        pltpu.sync_copy(hists[a].at[pl.ds(s * STRIPE, STRIPE)], zb)
        pltpu.sync_copy(zb, out_hbm.at[pl.ds(arr * NPAD + s * STRIPE, STRIPE)])


def _bincount8(e_pads):
    mesh = plsc.VectorSubcoreMesh(core_axis_name="c", subcore_axis_name="s")
    f = functools.partial(
        pl.kernel,
        mesh=mesh,
        out_type=jax.ShapeDtypeStruct((8 * NPAD,), jnp.float32),
        scratch_types=[
            pltpu.VMEM((NCHUNK, CHUNK), jnp.int32),
            pltpu.VMEM((CHUNK,), jnp.float32),
            pltpu.VMEM((STRIPE,), jnp.float32),
            pltpu.VMEM_SHARED((NPAD,), jnp.float32),
            pltpu.VMEM_SHARED((NPAD,), jnp.float32),
            pltpu.VMEM_SHARED((NPAD,), jnp.float32),
            pltpu.VMEM_SHARED((NPAD,), jnp.float32),
        ],
    )(_bincount8_kernel)
    return f(*e_pads).reshape(8, NPAD)


DSTAGE = 56
ACHUNK = 100
ANCH = EPAD // NTILE // ACHUNK
QCH = 32


def _agg_kernel(z_hbm, idxs_hbm, idxd_hbm, out_hbm,
                idxs, idxd, buf0, buf1, buf2, stage, acc,
                semg0, semg1, semg2, sems0, sems1, sems2):
    c = lax.axis_index("c")
    s = lax.axis_index("s")

    def zrow(r, carry):
        for col in range(4):
            stage[r, pl.ds(col * 16, 16)] = jnp.zeros((16,), jnp.float32)
        return carry

    lax.fori_loop(0, DSTAGE, zrow, None)
    for k in range(STRIPE // DSTAGE):
        pltpu.sync_copy(stage,
                        acc.at[pl.ds(s * STRIPE + k * DSTAGE, DSTAGE), :])
    plsc.subcore_barrier()

    bufs = (buf0, buf1, buf2)
    semg = (semg0, semg1, semg2)
    sems = (sems0, sems1, sems2)

    def start_g(j):
        pltpu.async_copy(z_hbm.at[idxs.at[j]], bufs[j % 3], semg[j % 3])

    def wait_g(j):
        pltpu.make_async_copy(z_hbm.at[idxs.at[j]], bufs[j % 3],
                              semg[j % 3]).wait()

    def start_s(j):
        pltpu.async_copy(bufs[j % 3], acc.at[idxd.at[j]], sems[j % 3],
                         add=True)

    def wait_s(j):
        pltpu.make_async_copy(bufs[j % 3], acc.at[idxd.at[j]],
                              sems[j % 3]).wait()

    for q in range(4):
        pltpu.sync_copy(idxs_hbm.at[(c * NTILE + s) * 4 + q], idxs)
        pltpu.sync_copy(idxd_hbm.at[s * 4 + q], idxd)
        start_g(0)
        start_g(1)
        for j in range(QCH):
            wait_g(j)
            start_s(j)
            if j + 2 < QCH:
                if j >= 1:
                    wait_s(j - 1)
                start_g(j + 2)
        wait_s(QCH - 3)
        wait_s(QCH - 2)
        wait_s(QCH - 1)
    plsc.subcore_barrier()
    for k in range(STRIPE // DSTAGE):
        pltpu.sync_copy(acc.at[pl.ds(s * STRIPE + k * DSTAGE, DSTAGE), :],
                        stage)
        pltpu.sync_copy(
            stage,
            out_hbm.at[pl.ds(s * STRIPE + k * DSTAGE, DSTAGE),
                       pl.ds(c * 64, 64)])


def _aggregate_rel(z2, idx_src, idx_dst):
    mesh = plsc.VectorSubcoreMesh(core_axis_name="c", subcore_axis_name="s")
    f = functools.partial(
        pl.kernel,
        mesh=mesh,
        out_type=jax.ShapeDtypeStruct((NPAD, 128), jnp.float32),
        scratch_types=[
            pltpu.VMEM((QCH, ACHUNK), jnp.int32),
            pltpu.VMEM((QCH, ACHUNK), jnp.int32),
            pltpu.VMEM((ACHUNK, 64), jnp.float32),
            pltpu.VMEM((ACHUNK, 64), jnp.float32),
            pltpu.VMEM((ACHUNK, 64), jnp.float32),
            pltpu.VMEM((DSTAGE, 64), jnp.float32),
            pltpu.VMEM_SHARED((NPAD, 64), jnp.float32),
            pltpu.SemaphoreType.DMA,
            pltpu.SemaphoreType.DMA,
            pltpu.SemaphoreType.DMA,
            pltpu.SemaphoreType.DMA,
            pltpu.SemaphoreType.DMA,
            pltpu.SemaphoreType.DMA,
        ],
        compiler_params=pltpu.CompilerParams(use_tc_tiling_on_sc=False),
    )(_agg_kernel)
    return f(z2, idx_src, idx_dst)


BR = 1000
BRD = 1000
_SRCTYPE = (0, 1, 1, 0)


def _make_z_kernel(rel):
    def _z_kernel(x_ref, deg_ref, w_ref, out_ref):
        h = jnp.dot(x_ref[...], w_ref[...],
                    preferred_element_type=jnp.float32)
        sc = lax.rsqrt(jnp.maximum(deg_ref[:, rel:rel + 1], 1.0))
        out_ref[...] = h * sc
    return _z_kernel


def _z_transform_rel(rel, x, deg_srcT, W):
    return pl.pallas_call(
        _make_z_kernel(rel),
        grid=(N // BR,),
        in_specs=[
            pl.BlockSpec((BR, DIN), lambda rb: (rb, 0)),
            pl.BlockSpec((BR, 4), lambda rb: (rb, 0)),
            pl.BlockSpec((DIN, DIN), lambda rb: (0, 0)),
        ],
        out_specs=pl.BlockSpec((BR, DIN), lambda rb: (rb, 0)),
        out_shape=jax.ShapeDtypeStruct((NPAD, DIN), jnp.float32),
    )(x, deg_srcT, W)


def _combine_kernel(a0_ref, a1_ref, a2_ref, a3_ref, deg_ref, bsum_ref,
                    hd_ref, ht_ref):
    s = [lax.rsqrt(jnp.maximum(deg_ref[:, r:r + 1], 1.0)) for r in range(4)]
    hd_ref[...] = a0_ref[...] * s[0] + a1_ref[...] * s[1] + bsum_ref[0:1]
    ht_ref[...] = a2_ref[...] * s[2] + a3_ref[...] * s[3] + bsum_ref[1:2]


def _combine(aggs, deg_dstT, bsum):
    return pl.pallas_call(
        _combine_kernel,
        grid=(N // BRD,),
        in_specs=[
            pl.BlockSpec((BRD, DIN), lambda rb: (rb, 0)),
            pl.BlockSpec((BRD, DIN), lambda rb: (rb, 0)),
            pl.BlockSpec((BRD, DIN), lambda rb: (rb, 0)),
            pl.BlockSpec((BRD, DIN), lambda rb: (rb, 0)),
            pl.BlockSpec((BRD, 4), lambda rb: (rb, 0)),
            pl.BlockSpec((2, DIN), lambda rb: (0, 0)),
        ],
        out_specs=[pl.BlockSpec((BRD, DIN), lambda rb: (rb, 0)),
                   pl.BlockSpec((BRD, DIN), lambda rb: (rb, 0))],
        out_shape=[jax.ShapeDtypeStruct((N, DIN), jnp.float32),
                   jax.ShapeDtypeStruct((N, DIN), jnp.float32)],
    )(*aggs, deg_dstT, bsum)


def kernel(x_d, x_t, edge_index_dd, edge_index_tt, edge_index_dt,
           edge_index_td, W_dd, b_dd, W_tt, b_tt, W_dt, b_dt):
    edges = [edge_index_dd, edge_index_td, edge_index_tt, edge_index_dt]
    npad_e = EPAD - E
    pad_idx = (N + (jnp.arange(npad_e) % 64)).astype(jnp.int32)
    pad2 = jnp.stack([pad_idx, pad_idx])
    e_pads = [jnp.concatenate([e, pad2], axis=1) for e in edges]

    hist = _bincount8(
        [e.reshape(2 * NTILE, NCHUNK, CHUNK) for e in e_pads])

    xs = (x_d, x_t)
    Ws = (W_dd, W_dt, W_tt, W_dt)
    deg_srcT = hist[0:4].T
    aggs = []
    for r in range(4):
        z = _z_transform_rel(r, xs[_SRCTYPE[r]], deg_srcT, Ws[r])
        z2 = z.reshape(2 * NPAD, 64)
        idx_src = jnp.stack(
            [2 * e_pads[r][0] + c for c in range(2)]).reshape(
                2 * NTILE * 4, QCH, ACHUNK)
        idx_dst = e_pads[r][1].reshape(NTILE * 4, QCH, ACHUNK)
        aggs.append(_aggregate_rel(z2, idx_src, idx_dst))

    deg_dstT = hist[4:8].T
    bsum = jnp.stack([b_dd + b_dt, b_tt + b_dt])
    h_d, h_t = _combine(aggs, deg_dstT, bsum)
    return (h_d, h_t)


# speedup vs baseline: 6.3523x; 1.0074x over previous
import functools

import jax
import jax.numpy as jnp
from jax import lax
from jax.experimental import pallas as pl
from jax.experimental.pallas import tpu as pltpu
from jax.experimental.pallas import tpu_sc as plsc

N = 25000
DIN = 128
E = 200000
NTILE = 16
NCORE = 2
NPAD = 25088
EPAD = 204800
CHUNK = 128
NCHUNK = EPAD // NTILE // CHUNK
STRIPE = NPAD // NTILE


def _bincount8_kernel(e0, e1, e2, e3, out_hbm, idxm, ones_v, zb,
                      h0, h1, h2, h3):
    c = lax.axis_index("c")
    s = lax.axis_index("s")
    hists = [h0, h1, h2, h3]
    for i in range(CHUNK // 16):
        ones_v[pl.ds(16 * i, 16)] = jnp.ones((16,), jnp.float32)
    for i in range(STRIPE // 16):
        zb[pl.ds(16 * i, 16)] = jnp.zeros((16,), jnp.float32)
    for a in range(4):
        pltpu.sync_copy(zb, hists[a].at[pl.ds(s * STRIPE, STRIPE)])
    plsc.subcore_barrier()
    for a, e in enumerate((e0, e1, e2, e3)):
        pltpu.sync_copy(e.at[c * NTILE + s], idxm)

        def body(j, carry, _h=hists[a]):
            pltpu.sync_copy(ones_v, _h.at[idxm.at[j]], add=True)
            return carry

        lax.fori_loop(0, NCHUNK, body, None)
    plsc.subcore_barrier()
    for a in range(4):
        arr = 4 * c + a
        pltpu.sync_copy(hists[a].at[pl.ds(s * STRIPE, STRIPE)], zb)
        pltpu.sync_copy(zb, out_hbm.at[pl.ds(arr * NPAD + s * STRIPE, STRIPE)])


def _bincount8(e_pads):
    mesh = plsc.VectorSubcoreMesh(core_axis_name="c", subcore_axis_name="s")
    f = functools.partial(
        pl.kernel,
        mesh=mesh,
        out_type=jax.ShapeDtypeStruct((8 * NPAD,), jnp.float32),
        scratch_types=[
            pltpu.VMEM((NCHUNK, CHUNK), jnp.int32),
            pltpu.VMEM((CHUNK,), jnp.float32),
            pltpu.VMEM((STRIPE,), jnp.float32),
            pltpu.VMEM_SHARED((NPAD,), jnp.float32),
            pltpu.VMEM_SHARED((NPAD,), jnp.float32),
            pltpu.VMEM_SHARED((NPAD,), jnp.float32),
            pltpu.VMEM_SHARED((NPAD,), jnp.float32),
        ],
    )(_bincount8_kernel)
    return f(*e_pads).reshape(8, NPAD)


DSTAGE = 56
ACHUNK = 100
ANCH = EPAD // NTILE // ACHUNK
QCH = 32


def _agg_kernel(z_hbm, idxs_hbm, idxd_hbm, out_hbm,
                idxs, idxd, buf0, buf1, buf2, stage, acc,
                semg0, semg1, semg2, sems0, sems1, sems2):
    c = lax.axis_index("c")
    s = lax.axis_index("s")

    def zrow(r, carry):
        for col in range(4):
            stage[r, pl.ds(col * 16, 16)] = jnp.zeros((16,), jnp.float32)
        return carry

    lax.fori_loop(0, DSTAGE, zrow, None)
    for k in range(STRIPE // DSTAGE):
        pltpu.sync_copy(stage,
                        acc.at[pl.ds(s * STRIPE + k * DSTAGE, DSTAGE), :])
    plsc.subcore_barrier()

    bufs = (buf0, buf1, buf2)
    semg = (semg0, semg1, semg2)
    sems = (sems0, sems1, sems2)

    def start_g(j):
        pltpu.async_copy(z_hbm.at[idxs.at[j]], bufs[j % 3], semg[j % 3])

    def wait_g(j):
        pltpu.make_async_copy(z_hbm.at[idxs.at[j]], bufs[j % 3],
                              semg[j % 3]).wait()

    def start_s(j):
        pltpu.async_copy(bufs[j % 3], acc.at[idxd.at[j]], sems[j % 3],
                         add=True)

    def wait_s(j):
        pltpu.make_async_copy(bufs[j % 3], acc.at[idxd.at[j]],
                              sems[j % 3]).wait()

    for q in range(4):
        pltpu.sync_copy(idxs_hbm.at[(c * NTILE + s) * 4 + q], idxs)
        pltpu.sync_copy(idxd_hbm.at[s * 4 + q], idxd)
        start_g(0)
        start_g(1)
        for j in range(QCH):
            wait_g(j)
            start_s(j)
            if j + 2 < QCH:
                if j >= 1:
                    wait_s(j - 1)
                start_g(j + 2)
        wait_s(QCH - 3)
        wait_s(QCH - 2)
        wait_s(QCH - 1)
    plsc.subcore_barrier()
    for k in range(STRIPE // DSTAGE):
        pltpu.sync_copy(acc.at[pl.ds(s * STRIPE + k * DSTAGE, DSTAGE), :],
                        stage)
        pltpu.sync_copy(
            stage,
            out_hbm.at[pl.ds(s * STRIPE + k * DSTAGE, DSTAGE),
                       pl.ds(c * 64, 64)])


def _aggregate_rel(z2, idx_src, idx_dst):
    mesh = plsc.VectorSubcoreMesh(core_axis_name="c", subcore_axis_name="s")
    f = functools.partial(
        pl.kernel,
        mesh=mesh,
        out_type=jax.ShapeDtypeStruct((NPAD, 128), jnp.float32),
        scratch_types=[
            pltpu.VMEM((QCH, ACHUNK), jnp.int32),
            pltpu.VMEM((QCH, ACHUNK), jnp.int32),
            pltpu.VMEM((ACHUNK, 64), jnp.float32),
            pltpu.VMEM((ACHUNK, 64), jnp.float32),
            pltpu.VMEM((ACHUNK, 64), jnp.float32),
            pltpu.VMEM((DSTAGE, 64), jnp.float32),
            pltpu.VMEM_SHARED((NPAD, 64), jnp.float32),
            pltpu.SemaphoreType.DMA,
            pltpu.SemaphoreType.DMA,
            pltpu.SemaphoreType.DMA,
            pltpu.SemaphoreType.DMA,
            pltpu.SemaphoreType.DMA,
            pltpu.SemaphoreType.DMA,
        ],
        compiler_params=pltpu.CompilerParams(use_tc_tiling_on_sc=False),
    )(_agg_kernel)
    return f(z2, idx_src, idx_dst)


BR = 1000
BRD = 1000
_SRCTYPE = (0, 1, 1, 0)


def _make_z_kernel(rel):
    def _z_kernel(x_ref, deg_ref, w_ref, out_ref):
        h = jnp.dot(x_ref[...], w_ref[...],
                    preferred_element_type=jnp.float32)
        sc = lax.rsqrt(jnp.maximum(deg_ref[:, rel:rel + 1], 1.0))
        out_ref[...] = h * sc
    return _z_kernel


def _z_transform_rel(rel, x, deg_srcT, W):
    return pl.pallas_call(
        _make_z_kernel(rel),
        grid=(N // BR,),
        in_specs=[
            pl.BlockSpec((BR, DIN), lambda rb: (rb, 0)),
            pl.BlockSpec((BR, 4), lambda rb: (rb, 0)),
            pl.BlockSpec((DIN, DIN), lambda rb: (0, 0)),
        ],
        out_specs=pl.BlockSpec((BR, DIN), lambda rb: (rb, 0)),
        out_shape=jax.ShapeDtypeStruct((NPAD, DIN), jnp.float32),
    )(x, deg_srcT, W)


def _make_combine_kernel(r0):
    def _combine_kernel(a0_ref, a1_ref, deg_ref, bias_ref, h_ref):
        s0 = lax.rsqrt(jnp.maximum(deg_ref[:, r0:r0 + 1], 1.0))
        s1 = lax.rsqrt(jnp.maximum(deg_ref[:, r0 + 1:r0 + 2], 1.0))
        h_ref[...] = a0_ref[...] * s0 + a1_ref[...] * s1 + bias_ref[0:1]
    return _combine_kernel


def _combine_pair(r0, a0, a1, deg_dstT, bias):
    return pl.pallas_call(
        _make_combine_kernel(r0),
        grid=(N // BRD,),
        in_specs=[
            pl.BlockSpec((BRD, DIN), lambda rb: (rb, 0)),
            pl.BlockSpec((BRD, DIN), lambda rb: (rb, 0)),
            pl.BlockSpec((BRD, 4), lambda rb: (rb, 0)),
            pl.BlockSpec((1, DIN), lambda rb: (0, 0)),
        ],
        out_specs=pl.BlockSpec((BRD, DIN), lambda rb: (rb, 0)),
        out_shape=jax.ShapeDtypeStruct((N, DIN), jnp.float32),
    )(a0, a1, deg_dstT, bias)


def kernel(x_d, x_t, edge_index_dd, edge_index_tt, edge_index_dt,
           edge_index_td, W_dd, b_dd, W_tt, b_tt, W_dt, b_dt):
    edges = [edge_index_dd, edge_index_td, edge_index_tt, edge_index_dt]
    npad_e = EPAD - E
    pad_idx = (N + (jnp.arange(npad_e) % 64)).astype(jnp.int32)
    pad2 = jnp.stack([pad_idx, pad_idx])
    e_pads = [jnp.concatenate([e, pad2], axis=1) for e in edges]

    hist = _bincount8(
        [e.reshape(2 * NTILE, NCHUNK, CHUNK) for e in e_pads])

    xs = (x_d, x_t)
    Ws = (W_dd, W_dt, W_tt, W_dt)
    deg_srcT = hist[0:4].T
    aggs = []
    for r in range(4):
        z = _z_transform_rel(r, xs[_SRCTYPE[r]], deg_srcT, Ws[r])
        z2 = z.reshape(2 * NPAD, 64)
        idx_src = jnp.stack(
            [2 * e_pads[r][0] + c for c in range(2)]).reshape(
                2 * NTILE * 4, QCH, ACHUNK)
        idx_dst = e_pads[r][1].reshape(NTILE * 4, QCH, ACHUNK)
        aggs.append(_aggregate_rel(z2, idx_src, idx_dst))

    deg_dstT = hist[4:8].T
    h_d = _combine_pair(0, aggs[0], aggs[1], deg_dstT,
                        (b_dd + b_dt)[None, :])
    h_t = _combine_pair(2, aggs[2], aggs[3], deg_dstT,
                        (b_tt + b_dt)[None, :])
    return (h_d, h_t)
